# Initial kernel scaffold; baseline (speedup 1.0000x reference)
import jax
import jax.numpy as jnp
from jax.experimental import pallas as pl


def kernel(input_ids, attention_mask, table):
    raise NotImplementedError("write your pallas kernel here")


# speedup vs baseline: 5.3547x; 5.3547x over previous
import functools

import jax
import jax.numpy as jnp
from jax import lax
from jax.experimental import pallas as pl
from jax.experimental.pallas import tpu as pltpu
from jax.experimental.pallas import tpu_sc as plsc

VOCAB = 100000
EMBED_DIM = 128
BATCH = 1024
SEQ_LEN = 200

NC = 2
NS = 16
LANES = 16
NW = NC * NS
TOTAL = BATCH * SEQ_LEN
PER_W = TOTAL // NW
CHUNK = 128
NCH = PER_W // CHUNK

@functools.cache
def _build_emb_lookup():
    mesh = plsc.VectorSubcoreMesh(core_axis_name="c", subcore_axis_name="s")
    return functools.partial(
        pl.kernel,
        mesh=mesh,
        out_type=jax.ShapeDtypeStruct((TOTAL, EMBED_DIM), jnp.float32),
        scratch_types=[
            pltpu.VMEM((NCH, CHUNK), jnp.int32),
            pltpu.VMEM((NCH, CHUNK), jnp.float32),
            pltpu.VMEM((CHUNK, EMBED_DIM), jnp.float32),
            pltpu.SemaphoreType.DMA,
        ],
    )(_emb_lookup)


def _lane_shuffle(x, perm):
    dnums = lax.GatherDimensionNumbers(
        offset_dims=(), collapsed_slice_dims=(0,), start_index_map=(0,))
    return lax.gather(x, perm.reshape(LANES, 1), dnums, (1,),
                      mode=lax.GatherScatterMode.PROMISE_IN_BOUNDS)


def _all_lanes_max(x):
    lane = jnp.arange(LANES, dtype=jnp.int32)
    for sh in (1, 2, 4, 8):
        x = jnp.maximum(x, _lane_shuffle(x, lane ^ sh))
    return x[0]


def _fix_chunk(g, idx_v, mask_v, rows_v):
    for j in range(CHUNK // LANES):
        iv = idx_v[g, pl.ds(j * LANES, LANES)]
        mv = mask_v[g, pl.ds(j * LANES, LANES)]
        badness = jnp.abs(mv - 1.0) + (1.0 - jnp.minimum(iv.astype(jnp.float32), 1.0))
        needs_fix = _all_lanes_max(badness) > 0.0

        @pl.when(needs_fix)
        def _fix_group(j=j, iv=iv, mv=mv):
            for l in range(LANES):
                row = j * LANES + l
                s = jnp.where(iv[l] == 0, 0.0, mv[l])

                @pl.when(s != 1.0)
                def _fix_row(row=row, s=s):
                    for k in range(EMBED_DIM // LANES):
                        sl = pl.ds(k * LANES, LANES)
                        rows_v[row, sl] = rows_v[row, sl] * s


def _emb_lookup(idx_hbm, mask_hbm, table_hbm, out_hbm, idx_v, mask_v, rows_v, sem):
    wid = lax.axis_index("s") * NC + lax.axis_index("c")
    pltpu.sync_copy(idx_hbm.at[wid], idx_v)
    pltpu.sync_copy(mask_hbm.at[wid], mask_v)

    def chunk_body(g, carry):
        pltpu.async_copy(table_hbm.at[idx_v.at[g]], rows_v, sem).wait()
        _fix_chunk(g, idx_v, mask_v, rows_v)

        pltpu.sync_copy(rows_v, out_hbm.at[pl.ds(wid * PER_W + g * CHUNK, CHUNK)])
        return carry

    lax.fori_loop(0, NCH, chunk_body, 0)


def kernel(input_ids, attention_mask, table):
    idx3 = input_ids.reshape(NW, NCH, CHUNK)
    mask3 = attention_mask.astype(jnp.float32).reshape(NW, NCH, CHUNK)
    out = _build_emb_lookup()(idx3, mask3, table)
    return out.reshape(BATCH, SEQ_LEN, EMBED_DIM)
